# baseline (speedup 1.0000x reference)
import functools

import jax
import jax.numpy as jnp
from jax import lax
from jax.experimental import pallas as pl
from jax.experimental.pallas import tpu as pltpu
from jax.experimental.pallas import tpu_sc as plsc

_B, _C, _T = 16, 512, 2048
_B_TC = 8
_B_SC = _B - _B_TC
_QUART = _C // 4
_CHUNK = 16
_NBUF = 3

_MASKED = [
    31, 35, 45, 63, 85, 99, 112, 114, 117, 121, 130, 139, 144, 148, 152,
    174, 176, 179, 188, 189, 197, 257, 263, 268, 272, 304, 309, 312, 315,
    318, 325, 356, 366, 398, 409, 410, 417, 429, 441, 446, 448, 462, 480,
    481, 487, 493, 495, 499, 501, 507, 509,
]


def _quarter_plan(q):
    lo = q * _QUART
    mset = set(c - lo for c in _MASKED if lo <= c < lo + _QUART)
    chunks = []
    for g in range(_QUART // _CHUNK):
        rows = range(g * _CHUNK, (g + 1) * _CHUNK)
        chunks.append(tuple(r - g * _CHUNK for r in rows if r in mset))
    return chunks


_PLANS = tuple(_quarter_plan(q) for q in range(4))

_mesh = plsc.VectorSubcoreMesh(core_axis_name="c", subcore_axis_name="s")


@functools.partial(
    pl.kernel,
    mesh=_mesh,
    out_type=jax.ShapeDtypeStruct((_B_SC * _C, _T), jnp.float32),
    scratch_types=(
        [pltpu.VMEM((_CHUNK, _T), jnp.float32) for _ in range(_NBUF)]
        + [pltpu.SemaphoreType.DMA for _ in range(2 * _NBUF)]
    ),
)
def _sc_mask_copy(x_hbm, out_hbm,
                  vb0, vb1, vb2,
                  si0, si1, si2, so0, so1, so2):
    s = lax.axis_index("s")
    c = lax.axis_index("c")
    batch = s // 2
    q = (s % 2) * 2 + c
    bufs = (vb0, vb1, vb2)
    isems = (si0, si1, si2)
    osems = (so0, so1, so2)

    for qq in range(4):
        chunks = _PLANS[qq]
        n_ck = len(chunks)

        @pl.when(q == qq)
        def _():
            base_in = (_B_TC + batch) * _C + qq * _QUART
            base_out = batch * _C + qq * _QUART

            def start_in(i):
                return pltpu.async_copy(
                    x_hbm.at[pl.ds(base_in + i * _CHUNK, _CHUNK)],
                    bufs[i % _NBUF],
                    isems[i % _NBUF],
                )

            in_h = {}
            out_h = {}
            out_waited = set()
            for j in range(min(_NBUF - 1, n_ck)):
                in_h[j] = start_in(j)
            for i in range(n_ck):
                nxt = _NBUF - 1 + i
                if nxt < n_ck:
                    if i > 0:
                        out_h[i - 1].wait()
                        out_waited.add(i - 1)
                    in_h[nxt] = start_in(nxt)
                in_h[i].wait()
                buf = bufs[i % _NBUF]
                for j in chunks[i]:
                    z = jnp.zeros((16,), jnp.float32)

                    def zero_row(k, carry, buf=buf, j=j, z=z):
                        buf[j, pl.ds(k * 64, 16)] = z
                        buf[j, pl.ds(k * 64 + 16, 16)] = z
                        buf[j, pl.ds(k * 64 + 32, 16)] = z
                        buf[j, pl.ds(k * 64 + 48, 16)] = z
                        return carry

                    lax.fori_loop(0, _T // 64, zero_row, 0)
                out_h[i] = pltpu.async_copy(
                    buf,
                    out_hbm.at[pl.ds(base_out + i * _CHUNK, _CHUNK)],
                    osems[i % _NBUF],
                )
            for i in range(n_ck):
                if i not in out_waited:
                    out_h[i].wait()


def _tc_body(x_ref, m_ref, o_ref):
    o_ref[...] = x_ref[...] * m_ref[...]


def _tc_mask(x, mask):
    return pl.pallas_call(
        _tc_body,
        grid=(_B_TC,),
        in_specs=[
            pl.BlockSpec((1, _C, _T), lambda b: (b, 0, 0)),
            pl.BlockSpec((_C, 1), lambda b: (0, 0)),
        ],
        out_specs=pl.BlockSpec((1, _C, _T), lambda b: (b, 0, 0)),
        out_shape=jax.ShapeDtypeStruct((_B_TC, _C, _T), jnp.float32),
    )(x, mask)


def kernel(x):
    B, C, T = x.shape
    x2 = x.reshape(B * C, T)
    sc_out = _sc_mask_copy(x2).reshape(_B_SC, C, T)
    mask = jnp.ones((C, 1), jnp.float32).at[jnp.array(_MASKED), :].set(0.0)
    tc_out = _tc_mask(x[:_B_TC], mask)
    return jnp.concatenate([tc_out, sc_out], axis=0)


# speedup vs baseline: 1.8803x; 1.8803x over previous
import functools

import jax
import jax.numpy as jnp
from jax import lax
from jax.experimental import pallas as pl
from jax.experimental.pallas import tpu as pltpu
from jax.experimental.pallas import tpu_sc as plsc

_B, _C, _T = 16, 512, 2048
_HALF = _C // 2
_CHUNK = 8
_NBUF = 3

_MASKED = [
    31, 35, 45, 63, 85, 99, 112, 114, 117, 121, 130, 139, 144, 148, 152,
    174, 176, 179, 188, 189, 197, 257, 263, 268, 272, 304, 309, 312, 315,
    318, 325, 356, 366, 398, 409, 410, 417, 429, 441, 446, 448, 462, 480,
    481, 487, 493, 495, 499, 501, 507, 509,
]


def _half_plan(h):
    lo = h * _HALF
    mset = set(c - lo for c in _MASKED if lo <= c < lo + _HALF)
    chunks = []
    for g in range(_HALF // _CHUNK):
        rows = range(g * _CHUNK, (g + 1) * _CHUNK)
        chunks.append(tuple(r - g * _CHUNK for r in rows if r in mset))
    return chunks


_PLANS = (_half_plan(0), _half_plan(1))

_mesh = plsc.VectorSubcoreMesh(core_axis_name="c", subcore_axis_name="s")


@functools.partial(
    pl.kernel,
    mesh=_mesh,
    out_type=jax.ShapeDtypeStruct((_B * _C, _T), jnp.float32),
    scratch_types=(
        [pltpu.VMEM((_CHUNK, _T), jnp.float32) for _ in range(_NBUF)]
        + [pltpu.VMEM_SHARED((16, _CHUNK, _T), jnp.float32) for _ in range(_NBUF)]
        + [pltpu.VMEM((1, _T), jnp.float32)]
        + [pltpu.SemaphoreType.DMA for _ in range(4 * _NBUF + 1)]
    ),
)
def _sc_mask_copy(x_hbm, out_hbm,
                  vb0, vb1, vb2, sb0, sb1, sb2, zrow,
                  vi0, vi1, vi2, vo0, vo1, vo2, si0, si1, si2, so0, so1, so2, sz):
    s = lax.axis_index("s")
    h = lax.axis_index("c")
    vbufs = (vb0, vb1, vb2)
    sbufs = (sb0, sb1, sb2)
    visems = (vi0, vi1, vi2)
    vosems = (vo0, vo1, vo2)
    sisems = (si0, si1, si2)
    sosems = (so0, so1, so2)

    z = jnp.zeros((16,), jnp.float32)

    def zinit(k, carry):
        zrow[0, pl.ds(k * 64, 16)] = z
        zrow[0, pl.ds(k * 64 + 16, 16)] = z
        zrow[0, pl.ds(k * 64 + 32, 16)] = z
        zrow[0, pl.ds(k * 64 + 48, 16)] = z
        return carry

    lax.fori_loop(0, _T // 64, zinit, 0)

    for hh in (0, 1):
        chunks = _PLANS[hh]
        n_ck = len(chunks)
        v_idx = [i for i in range(n_ck) if i % 2 == 0]
        s_idx = [i for i in range(n_ck) if i % 2 == 1]

        @pl.when(h == hh)
        def _():
            base = s * _C + hh * _HALF

            def hbm_rows(i):
                return pl.ds(base + i * _CHUNK, _CHUNK)

            def v_start_in(k):
                return pltpu.async_copy(
                    x_hbm.at[hbm_rows(v_idx[k])], vbufs[k % _NBUF],
                    visems[k % _NBUF])

            def s_start_in(k):
                return pltpu.async_copy(
                    x_hbm.at[hbm_rows(s_idx[k])], sbufs[k % _NBUF].at[s],
                    sisems[k % _NBUF])

            vin, vout, sin, sout = {}, {}, {}, {}
            v_w, s_w = set(), set()
            for k in range(min(_NBUF - 1, len(v_idx))):
                vin[k] = v_start_in(k)
            for k in range(min(_NBUF - 1, len(s_idx))):
                sin[k] = s_start_in(k)

            n_steps = max(len(v_idx), len(s_idx))
            for k in range(n_steps):
                if k < len(v_idx):
                    nk = _NBUF - 1 + k
                    if nk < len(v_idx):
                        if k > 0:
                            vout[k - 1].wait()
                            v_w.add(k - 1)
                        vin[nk] = v_start_in(nk)
                    vin[k].wait()
                    buf = vbufs[k % _NBUF]
                    for j in chunks[v_idx[k]]:
                        def zero_row(t, carry, buf=buf, j=j):
                            buf[j, pl.ds(t * 64, 16)] = z
                            buf[j, pl.ds(t * 64 + 16, 16)] = z
                            buf[j, pl.ds(t * 64 + 32, 16)] = z
                            buf[j, pl.ds(t * 64 + 48, 16)] = z
                            return carry
                        lax.fori_loop(0, _T // 64, zero_row, 0)
                    vout[k] = pltpu.async_copy(
                        buf, out_hbm.at[hbm_rows(v_idx[k])], vosems[k % _NBUF])
                if k < len(s_idx):
                    nk = _NBUF - 1 + k
                    if nk < len(s_idx):
                        if k > 0:
                            sout[k - 1].wait()
                            s_w.add(k - 1)
                        sin[nk] = s_start_in(nk)
                    sin[k].wait()
                    reg = sbufs[k % _NBUF].at[s]
                    for j in chunks[s_idx[k]]:
                        pltpu.async_copy(zrow, reg.at[pl.ds(j, 1)], sz).wait()
                    sout[k] = pltpu.async_copy(
                        reg, out_hbm.at[hbm_rows(s_idx[k])], sosems[k % _NBUF])
            for k in range(len(v_idx)):
                if k not in v_w:
                    vout[k].wait()
            for k in range(len(s_idx)):
                if k not in s_w:
                    sout[k].wait()


def kernel(x):
    B, C, T = x.shape
    x2 = x.reshape(B * C, T)
    out = _sc_mask_copy(x2)
    return out.reshape(B, C, T)


# speedup vs baseline: 2.9645x; 1.5766x over previous
import jax
import jax.numpy as jnp
from jax.experimental import pallas as pl

_MASKED = [
    31, 35, 45, 63, 85, 99, 112, 114, 117, 121, 130, 139, 144, 148, 152,
    174, 176, 179, 188, 189, 197, 257, 263, 268, 272, 304, 309, 312, 315,
    318, 325, 356, 366, 398, 409, 410, 417, 429, 441, 446, 448, 462, 480,
    481, 487, 493, 495, 499, 501, 507, 509,
]


def _body(x_ref, m_ref, o_ref):
    o_ref[...] = x_ref[...] * m_ref[...]


def kernel(x):
    B, C, T = x.shape
    mask = jnp.ones((1, C, 1), jnp.float32).at[0, jnp.array(_MASKED), :].set(0.0)
    return pl.pallas_call(
        _body,
        grid=(B // 2,),
        in_specs=[
            pl.BlockSpec((2, C, T), lambda b: (b, 0, 0)),
            pl.BlockSpec((1, C, 1), lambda b: (0, 0, 0)),
        ],
        out_specs=pl.BlockSpec((2, C, T), lambda b: (b, 0, 0)),
        out_shape=jax.ShapeDtypeStruct((B, C, T), x.dtype),
    )(x, mask)
